# baseline (speedup 1.0000x reference)
import functools

import jax
import jax.numpy as jnp
from jax import lax
from jax.experimental import pallas as pl
from jax.experimental.pallas import tpu as pltpu
from jax.experimental.pallas import tpu_sc as plsc

N = 10000
E = 320000
D = 128
G = 128
EPS = 1e-15
BN_EPS = 1e-5

NC, NS = 2, 16
NW = NC * NS
CHUNK = 128
CPT = 79
EPT = CPT * 2 * CHUNK
E_SC = NS * EPT
ZPT = 632
ACC_ROWS = NS * ZPT
OPT = 624

NPAD = 10240
RPT = NPAD // NW
PCH = 80
ACC_G = 144

BR = 1000


def _seg_body(ep, h2, zrows, out, idx2, rows, acc):
  c = lax.axis_index("c")
  s = lax.axis_index("s")
  pltpu.sync_copy(zrows.at[pl.ds(0, ZPT)], acc.at[pl.ds(s * ZPT, ZPT)])
  plsc.subcore_barrier()
  wid = c * NS + s

  def body(j, carry):
    cid = wid * CPT + j
    pltpu.sync_copy(ep.at[cid], idx2)
    pltpu.sync_copy(h2.at[idx2.at[0]], rows)
    pltpu.sync_copy(rows, acc.at[idx2.at[1]], add=True)
    pltpu.sync_copy(h2.at[idx2.at[2]], rows)
    pltpu.sync_copy(rows, acc.at[idx2.at[3]], add=True)
    return carry

  lax.fori_loop(0, CPT, body, 0)
  plsc.subcore_barrier()
  pltpu.sync_copy(acc.at[pl.ds(s * OPT, OPT)],
                  out.at[pl.ds(c * N + s * OPT, OPT)])

  @pl.when(s == NS - 1)
  def _():
    pltpu.sync_copy(acc.at[pl.ds(NS * OPT, N - NS * OPT)],
                    out.at[pl.ds(c * N + NS * OPT, N - NS * OPT)])


@functools.cache
def _get_seg_call():
  return pl.kernel(
      _seg_body,
      out_type=jax.ShapeDtypeStruct((2 * N, D), jnp.float32),
      mesh=plsc.VectorSubcoreMesh(core_axis_name="c", subcore_axis_name="s"),
      scratch_types=[
          pltpu.VMEM((4, CHUNK), jnp.int32),
          pltpu.VMEM((CHUNK, D), jnp.float32),
          pltpu.VMEM_SHARED((ACC_ROWS, D), jnp.float32),
      ],
  )


def _pool_body(totp, batchp, ones_hbm, zrows, out_s, out_c,
               idxb, rows, ones_v, acc_s, acc_c):
  c = lax.axis_index("c")
  s = lax.axis_index("s")
  pltpu.sync_copy(ones_hbm, ones_v)

  @pl.when(s < ACC_G // 16)
  def _():
    pltpu.sync_copy(zrows.at[pl.ds(0, 16)], acc_s.at[pl.ds(s * 16, 16)])
    pltpu.sync_copy(zrows.at[pl.ds(0, 16)], acc_c.at[pl.ds(s * 16, 16)])
  plsc.subcore_barrier()
  wid = c * NS + s

  def body(j, carry):
    base = (wid * 4 + j) * PCH
    pltpu.sync_copy(batchp.at[pl.ds(base, PCH)], idxb.at[0])
    pltpu.sync_copy(totp.at[pl.ds(base, PCH)], rows)
    pltpu.sync_copy(rows, acc_s.at[idxb.at[0]], add=True)
    pltpu.sync_copy(ones_v, acc_c.at[idxb.at[0]], add=True)
    return carry

  lax.fori_loop(0, 4, body, 0)
  plsc.subcore_barrier()

  @pl.when(s == 0)
  def _():
    pltpu.sync_copy(acc_s, out_s.at[c])
    pltpu.sync_copy(acc_c, out_c.at[c])


@functools.cache
def _get_pool_call():
  return pl.kernel(
      _pool_body,
      out_type=(jax.ShapeDtypeStruct((NC, ACC_G, D), jnp.float32),
                jax.ShapeDtypeStruct((NC, ACC_G, D), jnp.float32)),
      mesh=plsc.VectorSubcoreMesh(core_axis_name="c", subcore_axis_name="s"),
      scratch_types=[
          pltpu.VMEM((1, PCH), jnp.int32),
          pltpu.VMEM((PCH, D), jnp.float32),
          pltpu.VMEM((PCH, D), jnp.float32),
          pltpu.VMEM_SHARED((ACC_G, D), jnp.float32),
          pltpu.VMEM_SHARED((ACC_G, D), jnp.float32),
      ],
  )


def _layer0_body(h_ref, agg_ref, wr_ref, wroot_ref, p3_ref, out_ref):
  pre = (jnp.dot(agg_ref[...], wr_ref[...], preferred_element_type=jnp.float32)
         + jnp.dot(h_ref[...], wroot_ref[...],
                   preferred_element_type=jnp.float32)
         + p3_ref[0:1, :])
  out_ref[...] = jnp.maximum(pre * p3_ref[1:2, :] + p3_ref[2:3, :], 0.0)


def _layer_body(h_ref, agg_ref, wr_ref, wroot_ref, p3_ref, tin_ref,
                hout_ref, tout_ref):
  pre = (jnp.dot(agg_ref[...], wr_ref[...], preferred_element_type=jnp.float32)
         + jnp.dot(h_ref[...], wroot_ref[...],
                   preferred_element_type=jnp.float32)
         + p3_ref[0:1, :])
  h = jnp.maximum(pre * p3_ref[1:2, :] + p3_ref[2:3, :], 0.0)
  hout_ref[...] = h
  tout_ref[...] = tin_ref[...] + h


_row_spec = pl.BlockSpec((BR, D), lambda i: (i, 0))
_full_spec = pl.BlockSpec((D, D), lambda i: (0, 0))
_p3_spec = pl.BlockSpec((8, D), lambda i: (0, 0))

_layer0_call = pl.pallas_call(
    _layer0_body,
    grid=(2 * N // BR,),
    in_specs=[_row_spec, _row_spec, _full_spec, _full_spec, _p3_spec],
    out_specs=_row_spec,
    out_shape=jax.ShapeDtypeStruct((2 * N, D), jnp.float32),
)

_layer_call = pl.pallas_call(
    _layer_body,
    grid=(2 * N // BR,),
    in_specs=[_row_spec, _row_spec, _full_spec, _full_spec, _p3_spec,
              _row_spec],
    out_specs=[_row_spec, _row_spec],
    out_shape=[jax.ShapeDtypeStruct((2 * N, D), jnp.float32),
               jax.ShapeDtypeStruct((2 * N, D), jnp.float32)],
)


def _ws_body(ps_ref, pc_ref, ow_ref, p3_ref, dw_ref, ge_ref, ws_ref):
  sums = ps_ref[0, :G, :] + ps_ref[1, :G, :]
  cnt = pc_ref[0, :G, 0:1] + pc_ref[1, :G, 0:1]
  summary = sums / jnp.maximum(cnt, 1.0)
  pre = (jnp.dot(summary, ow_ref[...], preferred_element_type=jnp.float32)
         + p3_ref[0:1, :])
  ge_ref[...] = jnp.maximum(pre * p3_ref[1:2, :] + p3_ref[2:3, :], 0.0)
  ws_ref[...] = jnp.dot(dw_ref[...], summary,
                        preferred_element_type=jnp.float32)


_ws_call = pl.pallas_call(
    _ws_body,
    in_specs=[pl.BlockSpec((NC, ACC_G, D), lambda: (0, 0, 0)),
              pl.BlockSpec((NC, ACC_G, D), lambda: (0, 0, 0)),
              pl.BlockSpec((D, D), lambda: (0, 0)),
              pl.BlockSpec((8, D), lambda: (0, 0)),
              pl.BlockSpec((D, D), lambda: (0, 0))],
    out_specs=[pl.BlockSpec((G, D), lambda: (0, 0)),
               pl.BlockSpec((D, D), lambda: (0, 0))],
    out_shape=[jax.ShapeDtypeStruct((G, D), jnp.float32),
               jax.ShapeDtypeStruct((D, D), jnp.float32)],
)


def _loss_body(tot_ref, ws_ref, out_ref, accp, accn):
  i = pl.program_id(0)

  @pl.when(i == 0)
  def _():
    accp[0] = 0.0
    accn[0] = 0.0

  v = jnp.dot(tot_ref[...], ws_ref[...], preferred_element_type=jnp.float32)
  sig = 1.0 / (1.0 + jnp.exp(-v))
  is_pos = i < (N // BR)
  term = jnp.where(is_pos, jnp.log(sig + EPS), jnp.log(1.0 - sig + EPS))
  ssum = jnp.sum(term)
  accp[0] += jnp.where(is_pos, ssum, 0.0)
  accn[0] += jnp.where(is_pos, 0.0, ssum)

  @pl.when(i == 2 * N // BR - 1)
  def _():
    scale = 1.0 / (N * D)
    out_ref[...] = jnp.full((1, 1), -(accp[0] * scale) - (accn[0] * scale),
                            jnp.float32)


_loss_call = pl.pallas_call(
    _loss_body,
    grid=(2 * N // BR,),
    in_specs=[_row_spec, pl.BlockSpec((D, D), lambda i: (0, 0))],
    out_specs=pl.BlockSpec((1, 1), lambda i: (0, 0)),
    out_shape=jax.ShapeDtypeStruct((1, 1), jnp.float32),
    scratch_shapes=[pltpu.SMEM((1,), jnp.float32),
                    pltpu.SMEM((1,), jnp.float32)],
)


def _pack_edges(src, dst):
  pad = E_SC - E
  srcp = jnp.concatenate([src, jnp.zeros((pad,), jnp.int32)])
  dstp = jnp.concatenate([dst, jnp.full((pad,), N, jnp.int32)])
  s0 = srcp.reshape(NS * CPT, 2, CHUNK)
  d0 = dstp.reshape(NS * CPT, 2, CHUNK)

  def pack(sx):
    return jnp.stack([sx[:, 0], d0[:, 0], sx[:, 1], d0[:, 1]], axis=1)

  return jnp.concatenate([pack(s0), pack(s0 + N)], axis=0)


def _p3(b, g, beta):
  gs = g * (1.0 / jnp.sqrt(1.0 + BN_EPS))
  return jnp.concatenate([
      jnp.stack([b, gs, beta]),
      jnp.zeros((5, D), jnp.float32),
  ], axis=0)


def kernel(x, edge_index, batch, neg_x, params):
  p = params
  src = edge_index[0].astype(jnp.int32)
  dst = edge_index[1].astype(jnp.int32)
  ep = _pack_edges(src, dst)
  z_acc = jnp.zeros((ZPT, D), jnp.float32)
  z_pool = jnp.zeros((16, D), jnp.float32)
  ones_p = jnp.ones((PCH, D), jnp.float32)

  seg = _get_seg_call()
  h = jnp.concatenate([x, neg_x], axis=0)
  agg = seg(ep, h, z_acc)
  h = _layer0_call(h, agg, p['ingc_Wr'], p['ingc_Wroot'],
                   _p3(p['ingc_b'], p['inbn_g'], p['inbn_b']))
  tot = h
  for l in range(2):
    agg = seg(ep, h, z_acc)
    h, tot = _layer_call(h, agg, p[f'mid{l}_Wr'], p[f'mid{l}_Wroot'],
                         _p3(p[f'mid{l}_b'], p[f'bn{l}_g'], p[f'bn{l}_b']),
                         tot)

  totp = jnp.concatenate([tot[:N], jnp.zeros((NPAD - N, D), jnp.float32)],
                         axis=0)
  batchp = jnp.concatenate([batch.astype(jnp.int32),
                            jnp.full((NPAD - N,), G, jnp.int32)])
  pool_s, pool_c = _get_pool_call()(totp, batchp, ones_p, z_pool)
  graph_emb, ws = _ws_call(pool_s, pool_c, p['out_W'],
                           _p3(p['out_b'], p['dbn_g'], p['dbn_b']),
                           p['disc_W'])
  loss = _loss_call(tot, ws)[0, 0]
  return graph_emb, loss


# speedup vs baseline: 1.3126x; 1.3126x over previous
import functools

import jax
import jax.numpy as jnp
from jax import lax
from jax.experimental import pallas as pl
from jax.experimental.pallas import tpu as pltpu
from jax.experimental.pallas import tpu_sc as plsc

N = 10000
E = 320000
D = 128
G = 128
EPS = 1e-15
BN_EPS = 1e-5

NC, NS = 2, 16
NW = NC * NS
CHUNK = 128
CPT = 157
EPT = CPT * CHUNK
E_SC = NS * EPT
ZPT = 632
ACC_ROWS = NS * ZPT
OPT = 624

NPAD = 10240
RPT = NPAD // NW
PCH = 80
ACC_G = 144

BR = 1000


def _seg_body(ep, h2, zrows, out, epv, rows, acc):
  c = lax.axis_index("c")
  s = lax.axis_index("s")
  wid = c * NS + s
  pltpu.sync_copy(zrows.at[pl.ds(0, ZPT)], acc.at[pl.ds(s * ZPT, ZPT)])
  plsc.subcore_barrier()

  def body(j, carry):
    pltpu.sync_copy(h2.at[epv.at[j, 0]], rows)
    pltpu.sync_copy(rows, acc.at[epv.at[j, 1]], add=True)
    return carry

  HALF = 80
  pltpu.sync_copy(ep.at[pl.ds(wid * CPT, HALF)], epv)
  lax.fori_loop(0, HALF, body, 0)
  pltpu.sync_copy(ep.at[pl.ds(wid * CPT + HALF, CPT - HALF)],
                  epv.at[pl.ds(0, CPT - HALF)])
  lax.fori_loop(0, CPT - HALF, body, 0)
  plsc.subcore_barrier()
  pltpu.sync_copy(acc.at[pl.ds(s * OPT, OPT)],
                  out.at[pl.ds(c * N + s * OPT, OPT)])

  @pl.when(s == NS - 1)
  def _():
    pltpu.sync_copy(acc.at[pl.ds(NS * OPT, N - NS * OPT)],
                    out.at[pl.ds(c * N + NS * OPT, N - NS * OPT)])


@functools.cache
def _get_seg_call():
  return pl.kernel(
      _seg_body,
      out_type=jax.ShapeDtypeStruct((2 * N, D), jnp.float32),
      mesh=plsc.VectorSubcoreMesh(core_axis_name="c", subcore_axis_name="s"),
      scratch_types=[
          pltpu.VMEM((80, 2, CHUNK), jnp.int32),
          pltpu.VMEM((CHUNK, D), jnp.float32),
          pltpu.VMEM_SHARED((ACC_ROWS, D), jnp.float32),
      ],
  )


def _pool_body(totp, batchp, ones_hbm, zrows, out_s, out_c,
               idxb, rows, ones_v, acc_s, acc_c):
  c = lax.axis_index("c")
  s = lax.axis_index("s")
  pltpu.sync_copy(ones_hbm, ones_v)

  @pl.when(s < ACC_G // 16)
  def _():
    pltpu.sync_copy(zrows.at[pl.ds(0, 16)], acc_s.at[pl.ds(s * 16, 16)])
    pltpu.sync_copy(zrows.at[pl.ds(0, 16)], acc_c.at[pl.ds(s * 16, 16)])
  plsc.subcore_barrier()
  wid = c * NS + s

  def body(j, carry):
    base = (wid * 4 + j) * PCH
    pltpu.sync_copy(batchp.at[pl.ds(base, PCH)], idxb.at[0])
    pltpu.sync_copy(totp.at[pl.ds(base, PCH)], rows)
    pltpu.sync_copy(rows, acc_s.at[idxb.at[0]], add=True)
    pltpu.sync_copy(ones_v, acc_c.at[idxb.at[0]], add=True)
    return carry

  lax.fori_loop(0, 4, body, 0)
  plsc.subcore_barrier()

  @pl.when(s == 0)
  def _():
    pltpu.sync_copy(acc_s, out_s.at[c])
    pltpu.sync_copy(acc_c, out_c.at[c])


@functools.cache
def _get_pool_call():
  return pl.kernel(
      _pool_body,
      out_type=(jax.ShapeDtypeStruct((NC, ACC_G, D), jnp.float32),
                jax.ShapeDtypeStruct((NC, ACC_G, D), jnp.float32)),
      mesh=plsc.VectorSubcoreMesh(core_axis_name="c", subcore_axis_name="s"),
      scratch_types=[
          pltpu.VMEM((1, PCH), jnp.int32),
          pltpu.VMEM((PCH, D), jnp.float32),
          pltpu.VMEM((PCH, D), jnp.float32),
          pltpu.VMEM_SHARED((ACC_G, D), jnp.float32),
          pltpu.VMEM_SHARED((ACC_G, D), jnp.float32),
      ],
  )


def _layer0_body(h_ref, agg_ref, wr_ref, wroot_ref, p3_ref, out_ref):
  pre = (jnp.dot(agg_ref[...], wr_ref[...], preferred_element_type=jnp.float32)
         + jnp.dot(h_ref[...], wroot_ref[...],
                   preferred_element_type=jnp.float32)
         + p3_ref[0:1, :])
  out_ref[...] = jnp.maximum(pre * p3_ref[1:2, :] + p3_ref[2:3, :], 0.0)


def _layer_body(h_ref, agg_ref, wr_ref, wroot_ref, p3_ref, tin_ref,
                hout_ref, tout_ref):
  pre = (jnp.dot(agg_ref[...], wr_ref[...], preferred_element_type=jnp.float32)
         + jnp.dot(h_ref[...], wroot_ref[...],
                   preferred_element_type=jnp.float32)
         + p3_ref[0:1, :])
  h = jnp.maximum(pre * p3_ref[1:2, :] + p3_ref[2:3, :], 0.0)
  hout_ref[...] = h
  tout_ref[...] = tin_ref[...] + h


_row_spec = pl.BlockSpec((BR, D), lambda i: (i, 0))
_full_spec = pl.BlockSpec((D, D), lambda i: (0, 0))
_p3_spec = pl.BlockSpec((8, D), lambda i: (0, 0))

_layer0_call = pl.pallas_call(
    _layer0_body,
    grid=(2 * N // BR,),
    in_specs=[_row_spec, _row_spec, _full_spec, _full_spec, _p3_spec],
    out_specs=_row_spec,
    out_shape=jax.ShapeDtypeStruct((2 * N, D), jnp.float32),
)

_layer_call = pl.pallas_call(
    _layer_body,
    grid=(2 * N // BR,),
    in_specs=[_row_spec, _row_spec, _full_spec, _full_spec, _p3_spec,
              _row_spec],
    out_specs=[_row_spec, _row_spec],
    out_shape=[jax.ShapeDtypeStruct((2 * N, D), jnp.float32),
               jax.ShapeDtypeStruct((2 * N, D), jnp.float32)],
)


def _ws_body(ps_ref, pc_ref, ow_ref, p3_ref, dw_ref, ge_ref, ws_ref):
  sums = ps_ref[0, :G, :] + ps_ref[1, :G, :]
  cnt = pc_ref[0, :G, 0:1] + pc_ref[1, :G, 0:1]
  summary = sums / jnp.maximum(cnt, 1.0)
  pre = (jnp.dot(summary, ow_ref[...], preferred_element_type=jnp.float32)
         + p3_ref[0:1, :])
  ge_ref[...] = jnp.maximum(pre * p3_ref[1:2, :] + p3_ref[2:3, :], 0.0)
  ws_ref[...] = jnp.dot(dw_ref[...], summary,
                        preferred_element_type=jnp.float32)


_ws_call = pl.pallas_call(
    _ws_body,
    in_specs=[pl.BlockSpec((NC, ACC_G, D), lambda: (0, 0, 0)),
              pl.BlockSpec((NC, ACC_G, D), lambda: (0, 0, 0)),
              pl.BlockSpec((D, D), lambda: (0, 0)),
              pl.BlockSpec((8, D), lambda: (0, 0)),
              pl.BlockSpec((D, D), lambda: (0, 0))],
    out_specs=[pl.BlockSpec((G, D), lambda: (0, 0)),
               pl.BlockSpec((D, D), lambda: (0, 0))],
    out_shape=[jax.ShapeDtypeStruct((G, D), jnp.float32),
               jax.ShapeDtypeStruct((D, D), jnp.float32)],
)


def _loss_body(tot_ref, ws_ref, out_ref, accp, accn):
  i = pl.program_id(0)

  @pl.when(i == 0)
  def _():
    accp[0] = 0.0
    accn[0] = 0.0

  v = jnp.dot(tot_ref[...], ws_ref[...], preferred_element_type=jnp.float32)
  sig = 1.0 / (1.0 + jnp.exp(-v))
  is_pos = i < (N // BR)
  term = jnp.where(is_pos, jnp.log(sig + EPS), jnp.log(1.0 - sig + EPS))
  ssum = jnp.sum(term)
  accp[0] += jnp.where(is_pos, ssum, 0.0)
  accn[0] += jnp.where(is_pos, 0.0, ssum)

  @pl.when(i == 2 * N // BR - 1)
  def _():
    scale = 1.0 / (N * D)
    out_ref[...] = jnp.full((1, 1), -(accp[0] * scale) - (accn[0] * scale),
                            jnp.float32)


_loss_call = pl.pallas_call(
    _loss_body,
    grid=(2 * N // BR,),
    in_specs=[_row_spec, pl.BlockSpec((D, D), lambda i: (0, 0))],
    out_specs=pl.BlockSpec((1, 1), lambda i: (0, 0)),
    out_shape=jax.ShapeDtypeStruct((1, 1), jnp.float32),
    scratch_shapes=[pltpu.SMEM((1,), jnp.float32),
                    pltpu.SMEM((1,), jnp.float32)],
)


def _pack_edges(src, dst):
  pad = E_SC - E
  srcp = jnp.concatenate([src, jnp.zeros((pad,), jnp.int32)])
  dstp = jnp.concatenate([dst, jnp.full((pad,), N, jnp.int32)])
  s0 = srcp.reshape(NS * CPT, CHUNK)
  d0 = dstp.reshape(NS * CPT, CHUNK)
  return jnp.concatenate([
      jnp.stack([s0, d0], axis=1),
      jnp.stack([s0 + N, d0], axis=1),
  ], axis=0)


def _p3(b, g, beta):
  gs = g * (1.0 / jnp.sqrt(1.0 + BN_EPS))
  return jnp.concatenate([
      jnp.stack([b, gs, beta]),
      jnp.zeros((5, D), jnp.float32),
  ], axis=0)


def kernel(x, edge_index, batch, neg_x, params):
  p = params
  src = edge_index[0].astype(jnp.int32)
  dst = edge_index[1].astype(jnp.int32)
  ep = _pack_edges(src, dst)
  z_acc = jnp.zeros((ZPT, D), jnp.float32)
  z_pool = jnp.zeros((16, D), jnp.float32)
  ones_p = jnp.ones((PCH, D), jnp.float32)

  seg = _get_seg_call()
  h = jnp.concatenate([x, neg_x], axis=0)
  agg = seg(ep, h, z_acc)
  h = _layer0_call(h, agg, p['ingc_Wr'], p['ingc_Wroot'],
                   _p3(p['ingc_b'], p['inbn_g'], p['inbn_b']))
  tot = h
  for l in range(2):
    agg = seg(ep, h, z_acc)
    h, tot = _layer_call(h, agg, p[f'mid{l}_Wr'], p[f'mid{l}_Wroot'],
                         _p3(p[f'mid{l}_b'], p[f'bn{l}_g'], p[f'bn{l}_b']),
                         tot)

  totp = jnp.concatenate([tot[:N], jnp.zeros((NPAD - N, D), jnp.float32)],
                         axis=0)
  batchp = jnp.concatenate([batch.astype(jnp.int32),
                            jnp.full((NPAD - N,), G, jnp.int32)])
  pool_s, pool_c = _get_pool_call()(totp, batchp, ones_p, z_pool)
  graph_emb, ws = _ws_call(pool_s, pool_c, p['out_W'],
                           _p3(p['out_b'], p['dbn_g'], p['dbn_b']),
                           p['disc_W'])
  loss = _loss_call(tot, ws)[0, 0]
  return graph_emb, loss


# speedup vs baseline: 1.4905x; 1.1356x over previous
import functools

import jax
import jax.numpy as jnp
from jax import lax
from jax.experimental import pallas as pl
from jax.experimental.pallas import tpu as pltpu
from jax.experimental.pallas import tpu_sc as plsc

N = 10000
E = 320000
D = 128
G = 128
EPS = 1e-15
BN_EPS = 1e-5

NC, NS = 2, 16
NW = NC * NS
CHUNK = 128
CPT = 157
EPT = CPT * CHUNK
E_SC = NS * EPT
ZPT = 632
ACC_ROWS = NS * ZPT
OPT = 624

NPAD = 10240
RPT = NPAD // NW
PCH = 80
ACC_G = 144

BR = 1000


def _seg_body(ep, h2, zrows, out, epv, rows_a, rows_b,
              gs_a, gs_b, ss_a, ss_b, acc):
  c = lax.axis_index("c")
  s = lax.axis_index("s")
  wid = c * NS + s
  pltpu.sync_copy(zrows.at[pl.ds(0, ZPT)], acc.at[pl.ds(s * ZPT, ZPT)])
  plsc.subcore_barrier()

  def pair(p, carry):
    @pl.when(p > 0)
    def _():
      pltpu.make_async_copy(rows_a, acc.at[epv.at[2 * p, 1]], ss_a).wait()
    pltpu.async_copy(h2.at[epv.at[2 * p, 0]], rows_a, gs_a)

    @pl.when(p > 0)
    def _():
      pltpu.make_async_copy(rows_b, acc.at[epv.at[2 * p, 1]], ss_b).wait()
    pltpu.async_copy(h2.at[epv.at[2 * p + 1, 0]], rows_b, gs_b)
    pltpu.make_async_copy(h2.at[epv.at[2 * p, 0]], rows_a, gs_a).wait()
    pltpu.async_copy(rows_a, acc.at[epv.at[2 * p, 1]], ss_a, add=True)
    pltpu.make_async_copy(h2.at[epv.at[2 * p + 1, 0]], rows_b, gs_b).wait()
    pltpu.async_copy(rows_b, acc.at[epv.at[2 * p + 1, 1]], ss_b, add=True)
    return carry

  def drain():
    pltpu.make_async_copy(rows_a, acc.at[epv.at[0, 1]], ss_a).wait()
    pltpu.make_async_copy(rows_b, acc.at[epv.at[0, 1]], ss_b).wait()

  def body(j, carry):
    pltpu.sync_copy(h2.at[epv.at[j, 0]], rows_a)
    pltpu.sync_copy(rows_a, acc.at[epv.at[j, 1]], add=True)
    return carry

  pltpu.sync_copy(ep.at[pl.ds(wid * CPT, 64)], epv)
  lax.fori_loop(0, 32, pair, 0)
  drain()
  pltpu.sync_copy(ep.at[pl.ds(wid * CPT + 64, 64)], epv)
  lax.fori_loop(0, 32, pair, 0)
  drain()
  pltpu.sync_copy(ep.at[pl.ds(wid * CPT + 128, CPT - 128)],
                  epv.at[pl.ds(0, CPT - 128)])
  lax.fori_loop(0, 14, pair, 0)
  drain()
  lax.fori_loop(28, CPT - 128, body, 0)
  plsc.subcore_barrier()
  pltpu.sync_copy(acc.at[pl.ds(s * OPT, OPT)],
                  out.at[pl.ds(c * N + s * OPT, OPT)])

  @pl.when(s == NS - 1)
  def _():
    pltpu.sync_copy(acc.at[pl.ds(NS * OPT, N - NS * OPT)],
                    out.at[pl.ds(c * N + NS * OPT, N - NS * OPT)])


@functools.cache
def _get_seg_call():
  return pl.kernel(
      _seg_body,
      out_type=jax.ShapeDtypeStruct((2 * N, D), jnp.float32),
      mesh=plsc.VectorSubcoreMesh(core_axis_name="c", subcore_axis_name="s"),
      scratch_types=[
          pltpu.VMEM((64, 2, CHUNK), jnp.int32),
          pltpu.VMEM((CHUNK, D), jnp.float32),
          pltpu.VMEM((CHUNK, D), jnp.float32),
          pltpu.SemaphoreType.DMA,
          pltpu.SemaphoreType.DMA,
          pltpu.SemaphoreType.DMA,
          pltpu.SemaphoreType.DMA,
          pltpu.VMEM_SHARED((ACC_ROWS, D), jnp.float32),
      ],
  )


def _pool_body(totp, batchp, ones_hbm, zrows, out_s, out_c,
               idxb, rows, ones_v, acc_s, acc_c):
  c = lax.axis_index("c")
  s = lax.axis_index("s")
  pltpu.sync_copy(ones_hbm, ones_v)

  @pl.when(s < ACC_G // 16)
  def _():
    pltpu.sync_copy(zrows.at[pl.ds(0, 16)], acc_s.at[pl.ds(s * 16, 16)])
    pltpu.sync_copy(zrows.at[pl.ds(0, 16)], acc_c.at[pl.ds(s * 16, 16)])
  plsc.subcore_barrier()
  wid = c * NS + s

  def body(j, carry):
    base = (wid * 4 + j) * PCH
    pltpu.sync_copy(batchp.at[pl.ds(base, PCH)], idxb.at[0])
    pltpu.sync_copy(totp.at[pl.ds(base, PCH)], rows)
    pltpu.sync_copy(rows, acc_s.at[idxb.at[0]], add=True)
    pltpu.sync_copy(ones_v, acc_c.at[idxb.at[0]], add=True)
    return carry

  lax.fori_loop(0, 4, body, 0)
  plsc.subcore_barrier()

  @pl.when(s == 0)
  def _():
    pltpu.sync_copy(acc_s, out_s.at[c])
    pltpu.sync_copy(acc_c, out_c.at[c])


@functools.cache
def _get_pool_call():
  return pl.kernel(
      _pool_body,
      out_type=(jax.ShapeDtypeStruct((NC, ACC_G, D), jnp.float32),
                jax.ShapeDtypeStruct((NC, ACC_G, D), jnp.float32)),
      mesh=plsc.VectorSubcoreMesh(core_axis_name="c", subcore_axis_name="s"),
      scratch_types=[
          pltpu.VMEM((1, PCH), jnp.int32),
          pltpu.VMEM((PCH, D), jnp.float32),
          pltpu.VMEM((PCH, D), jnp.float32),
          pltpu.VMEM_SHARED((ACC_G, D), jnp.float32),
          pltpu.VMEM_SHARED((ACC_G, D), jnp.float32),
      ],
  )


def _layer0_body(h_ref, agg_ref, wr_ref, wroot_ref, p3_ref, out_ref):
  pre = (jnp.dot(agg_ref[...], wr_ref[...], preferred_element_type=jnp.float32)
         + jnp.dot(h_ref[...], wroot_ref[...],
                   preferred_element_type=jnp.float32)
         + p3_ref[0:1, :])
  out_ref[...] = jnp.maximum(pre * p3_ref[1:2, :] + p3_ref[2:3, :], 0.0)


def _layer_body(h_ref, agg_ref, wr_ref, wroot_ref, p3_ref, tin_ref,
                hout_ref, tout_ref):
  pre = (jnp.dot(agg_ref[...], wr_ref[...], preferred_element_type=jnp.float32)
         + jnp.dot(h_ref[...], wroot_ref[...],
                   preferred_element_type=jnp.float32)
         + p3_ref[0:1, :])
  h = jnp.maximum(pre * p3_ref[1:2, :] + p3_ref[2:3, :], 0.0)
  hout_ref[...] = h
  tout_ref[...] = tin_ref[...] + h


_row_spec = pl.BlockSpec((BR, D), lambda i: (i, 0))
_full_spec = pl.BlockSpec((D, D), lambda i: (0, 0))
_p3_spec = pl.BlockSpec((8, D), lambda i: (0, 0))

_layer0_call = pl.pallas_call(
    _layer0_body,
    grid=(2 * N // BR,),
    in_specs=[_row_spec, _row_spec, _full_spec, _full_spec, _p3_spec],
    out_specs=_row_spec,
    out_shape=jax.ShapeDtypeStruct((2 * N, D), jnp.float32),
)

_layer_call = pl.pallas_call(
    _layer_body,
    grid=(2 * N // BR,),
    in_specs=[_row_spec, _row_spec, _full_spec, _full_spec, _p3_spec,
              _row_spec],
    out_specs=[_row_spec, _row_spec],
    out_shape=[jax.ShapeDtypeStruct((2 * N, D), jnp.float32),
               jax.ShapeDtypeStruct((2 * N, D), jnp.float32)],
)


def _ws_body(ps_ref, pc_ref, ow_ref, p3_ref, dw_ref, ge_ref, ws_ref):
  sums = ps_ref[0, :G, :] + ps_ref[1, :G, :]
  cnt = pc_ref[0, :G, 0:1] + pc_ref[1, :G, 0:1]
  summary = sums / jnp.maximum(cnt, 1.0)
  pre = (jnp.dot(summary, ow_ref[...], preferred_element_type=jnp.float32)
         + p3_ref[0:1, :])
  ge_ref[...] = jnp.maximum(pre * p3_ref[1:2, :] + p3_ref[2:3, :], 0.0)
  ws_ref[...] = jnp.dot(dw_ref[...], summary,
                        preferred_element_type=jnp.float32)


_ws_call = pl.pallas_call(
    _ws_body,
    in_specs=[pl.BlockSpec((NC, ACC_G, D), lambda: (0, 0, 0)),
              pl.BlockSpec((NC, ACC_G, D), lambda: (0, 0, 0)),
              pl.BlockSpec((D, D), lambda: (0, 0)),
              pl.BlockSpec((8, D), lambda: (0, 0)),
              pl.BlockSpec((D, D), lambda: (0, 0))],
    out_specs=[pl.BlockSpec((G, D), lambda: (0, 0)),
               pl.BlockSpec((D, D), lambda: (0, 0))],
    out_shape=[jax.ShapeDtypeStruct((G, D), jnp.float32),
               jax.ShapeDtypeStruct((D, D), jnp.float32)],
)


def _loss_body(tot_ref, ws_ref, out_ref, accp, accn):
  i = pl.program_id(0)

  @pl.when(i == 0)
  def _():
    accp[0] = 0.0
    accn[0] = 0.0

  v = jnp.dot(tot_ref[...], ws_ref[...], preferred_element_type=jnp.float32)
  sig = 1.0 / (1.0 + jnp.exp(-v))
  is_pos = i < (N // BR)
  term = jnp.where(is_pos, jnp.log(sig + EPS), jnp.log(1.0 - sig + EPS))
  ssum = jnp.sum(term)
  accp[0] += jnp.where(is_pos, ssum, 0.0)
  accn[0] += jnp.where(is_pos, 0.0, ssum)

  @pl.when(i == 2 * N // BR - 1)
  def _():
    scale = 1.0 / (N * D)
    out_ref[...] = jnp.full((1, 1), -(accp[0] * scale) - (accn[0] * scale),
                            jnp.float32)


_loss_call = pl.pallas_call(
    _loss_body,
    grid=(2 * N // BR,),
    in_specs=[_row_spec, pl.BlockSpec((D, D), lambda i: (0, 0))],
    out_specs=pl.BlockSpec((1, 1), lambda i: (0, 0)),
    out_shape=jax.ShapeDtypeStruct((1, 1), jnp.float32),
    scratch_shapes=[pltpu.SMEM((1,), jnp.float32),
                    pltpu.SMEM((1,), jnp.float32)],
)


def _pack_edges(src, dst):
  pad = E_SC - E
  srcp = jnp.concatenate([src, jnp.zeros((pad,), jnp.int32)])
  dstp = jnp.concatenate([dst, jnp.full((pad,), N, jnp.int32)])
  s0 = srcp.reshape(NS * CPT, CHUNK)
  d0 = dstp.reshape(NS * CPT, CHUNK)
  return jnp.concatenate([
      jnp.stack([s0, d0], axis=1),
      jnp.stack([s0 + N, d0], axis=1),
  ], axis=0)


def _p3(b, g, beta):
  gs = g * (1.0 / jnp.sqrt(1.0 + BN_EPS))
  return jnp.concatenate([
      jnp.stack([b, gs, beta]),
      jnp.zeros((5, D), jnp.float32),
  ], axis=0)


def kernel(x, edge_index, batch, neg_x, params):
  p = params
  src = edge_index[0].astype(jnp.int32)
  dst = edge_index[1].astype(jnp.int32)
  ep = _pack_edges(src, dst)
  z_acc = jnp.zeros((ZPT, D), jnp.float32)
  z_pool = jnp.zeros((16, D), jnp.float32)
  ones_p = jnp.ones((PCH, D), jnp.float32)

  seg = _get_seg_call()
  h = jnp.concatenate([x, neg_x], axis=0)
  agg = seg(ep, h, z_acc)
  h = _layer0_call(h, agg, p['ingc_Wr'], p['ingc_Wroot'],
                   _p3(p['ingc_b'], p['inbn_g'], p['inbn_b']))
  tot = h
  for l in range(2):
    agg = seg(ep, h, z_acc)
    h, tot = _layer_call(h, agg, p[f'mid{l}_Wr'], p[f'mid{l}_Wroot'],
                         _p3(p[f'mid{l}_b'], p[f'bn{l}_g'], p[f'bn{l}_b']),
                         tot)

  totp = jnp.concatenate([tot[:N], jnp.zeros((NPAD - N, D), jnp.float32)],
                         axis=0)
  batchp = jnp.concatenate([batch.astype(jnp.int32),
                            jnp.full((NPAD - N,), G, jnp.int32)])
  pool_s, pool_c = _get_pool_call()(totp, batchp, ones_p, z_pool)
  graph_emb, ws = _ws_call(pool_s, pool_c, p['out_W'],
                           _p3(p['out_b'], p['dbn_g'], p['dbn_b']),
                           p['disc_W'])
  loss = _loss_call(tot, ws)[0, 0]
  return graph_emb, loss
